# baseline (speedup 1.0000x reference)
import jax
import jax.numpy as jnp
from jax.experimental import pallas as pl

H = 64
OUT_DIM = 2
N_PRED = 12


def _graph_conv(x, src, dst, W, b, ods, ids):
    n = x.shape[0]
    h = x * ods[:, None]
    if W.shape[0] > W.shape[1]:
        h = h @ W
    agg = jax.ops.segment_sum(h[src], dst, num_segments=n)
    agg = agg * ids[:, None]
    if W.shape[0] <= W.shape[1]:
        agg = agg @ W
    return agg + b


def _cell(x, state, src, dst, Wru, bru, Wc, bc, ods, ids):
    ru = jax.nn.sigmoid(_graph_conv(jnp.concatenate([x, state], -1), src, dst, Wru, bru, ods, ids))
    r = ru[:, :H]
    u = ru[:, H:]
    c = jnp.tanh(_graph_conv(jnp.concatenate([x, r * state], -1), src, dst, Wc, bc, ods, ids))
    new_state = u * state + (1.0 - u) * c
    return new_state, new_state


def _proj_kernel(x_ref, w_ref, b_ref, o_ref):
    o_ref[...] = x_ref[...] @ w_ref[...] + b_ref[...]


def _proj(x, W, b):
    n = x.shape[0]
    return pl.pallas_call(
        _proj_kernel,
        out_shape=jax.ShapeDtypeStruct((n, W.shape[1]), x.dtype),
    )(x, W, b[None, :])


def kernel(inputs, edge_index, batch_seen, enc0_Wru, enc0_bru, enc0_Wc, enc0_bc, enc1_Wru, enc1_bru, enc1_Wc, enc1_bc, dec0_Wru, dec0_bru, dec0_Wc, dec0_bc, dec1_Wru, dec1_bru, dec1_Wc, dec1_bc, out_W, out_b):
    src = edge_index[0]
    dst = edge_index[1]
    n = inputs.shape[0]
    t = inputs.shape[1]
    ones = jnp.ones(src.shape[0], dtype=inputs.dtype)
    out_deg = jnp.clip(jax.ops.segment_sum(ones, src, num_segments=n), 1.0)
    in_deg = jnp.clip(jax.ops.segment_sum(ones, dst, num_segments=n), 1.0)
    ods = out_deg ** -0.5
    ids = in_deg ** -0.5

    enc_params = [(enc0_Wru, enc0_bru, enc0_Wc, enc0_bc), (enc1_Wru, enc1_bru, enc1_Wc, enc1_bc)]
    states = [jnp.zeros((n, H), inputs.dtype) for _ in range(2)]
    xs = [inputs[:, i, :] for i in range(t)]
    for i_layer in range(2):
        Wru, bru, Wc, bc = enc_params[i_layer]
        st = states[i_layer]
        for i_t in range(t):
            xs[i_t], st = _cell(xs[i_t], st, src, dst, Wru, bru, Wc, bc, ods, ids)
        states[i_layer] = st

    dec_params = [(dec0_Wru, dec0_bru, dec0_Wc, dec0_bc), (dec1_Wru, dec1_bru, dec1_Wc, dec1_bc)]
    x = jnp.zeros((n, OUT_DIM), inputs.dtype)
    outs = []
    for i_t in range(N_PRED):
        for i_layer in range(2):
            Wru, bru, Wc, bc = dec_params[i_layer]
            x, states[i_layer] = _cell(x, states[i_layer], src, dst, Wru, bru, Wc, bc, ods, ids)
        x = _proj(x, out_W, out_b)
        outs.append(x)
    return jnp.stack(outs, 1)
